# Initial kernel scaffold; baseline (speedup 1.0000x reference)
import jax
import jax.numpy as jnp
from jax.experimental import pallas as pl


def kernel(p_cat, img_size):
    raise NotImplementedError("write your pallas kernel here")


# speedup vs baseline: 1.8365x; 1.8365x over previous
import jax
import jax.numpy as jnp
from jax import lax
from jax.experimental import pallas as pl
from jax.experimental.pallas import tpu as pltpu

_NA = 4
_NC = 1
_EMB = 512
_ANCHORS_W = (32.0, 45.0, 64.0, 90.0)
_ANCHORS_H = (96.0, 135.0, 192.0, 273.0)
_NB, _NGH, _NGW = 8, 38, 68
_NS = _NGH * _NGW
_SCH = 136
_NSB = _NS // _SCH
_BOX_CH = _NA * (_NC + 5)
_OUT_CH = 4 + 1 + _NC + _EMB


def _select_anchor(a, vals):
    out = jnp.float32(vals[0])
    for i in range(1, _NA):
        out = jnp.where(a == i, jnp.float32(vals[i]), out)
    return out


def _body(stride_ref, box_ref, emb_ref, out_ref):
    s = pl.program_id(1)
    a = pl.program_id(2)
    stride = stride_ref[0, 0]
    aw = _select_anchor(a, _ANCHORS_W)
    ah = _select_anchor(a, _ANCHORS_H)
    pw = aw / stride
    ph = ah / stride

    xb = box_ref[0, 0, 0]
    dx = xb[:, 0:1]
    dy = xb[:, 1:2]
    dw = xb[:, 2:3]
    dh = xb[:, 3:4]
    c0 = xb[:, 4:5]
    c1 = xb[:, 5:6]

    idx = s * _SCH + lax.broadcasted_iota(jnp.int32, (_SCH, 1), 0)
    px = (idx % _NGW).astype(jnp.float32)
    py = (idx // _NGW).astype(jnp.float32)

    gx = (pw * dx + px) * stride
    gy = (ph * dy + py) * stride
    gw = pw * stride * jnp.exp(dw)
    gh = ph * stride * jnp.exp(dh)
    conf = jax.nn.sigmoid(c1 - c0)
    cls = jnp.zeros_like(conf)
    head = jnp.concatenate([gx, gy, gw, gh, conf, cls], axis=1)

    emb = emb_ref[0, 0]
    ssq = jnp.sum(emb * emb, axis=1, keepdims=True)
    inv = 1.0 / jnp.maximum(jnp.sqrt(ssq), 1e-12)

    out_ref[0, :, 0:6] = head
    out_ref[0, :, 6:_OUT_CH] = emb * inv


def kernel(p_cat, img_size):
    nB = p_cat.shape[0]
    xf = p_cat.reshape(nB, _BOX_CH + _EMB, _NS)
    box_t = (
        xf[:, :_BOX_CH, :]
        .reshape(nB, _NA, _NC + 5, _NS)
        .transpose(0, 1, 3, 2)
        .reshape(nB, _NA, _NSB, _SCH, _NC + 5)
    )
    emb_t = (
        xf[:, _BOX_CH:, :]
        .transpose(0, 2, 1)
        .reshape(nB, _NSB, _SCH, _EMB)
    )
    stride = (jnp.asarray(img_size[0], jnp.float32) / _NGW).reshape(1, 1)

    out = pl.pallas_call(
        _body,
        grid=(nB, _NSB, _NA),
        in_specs=[
            pl.BlockSpec(memory_space=pltpu.SMEM),
            pl.BlockSpec(
                (1, 1, 1, _SCH, _NC + 5), lambda b, s, a: (b, a, s, 0, 0)
            ),
            pl.BlockSpec((1, 1, _SCH, _EMB), lambda b, s, a: (b, s, 0, 0)),
        ],
        out_specs=pl.BlockSpec(
            (1, _SCH, _OUT_CH), lambda b, s, a: (b, a * _NSB + s, 0)
        ),
        out_shape=jax.ShapeDtypeStruct((nB, _NA * _NS, _OUT_CH), jnp.float32),
    )(stride, box_t, emb_t)
    return out
